# Initial kernel scaffold; baseline (speedup 1.0000x reference)
import jax
import jax.numpy as jnp
from jax.experimental import pallas as pl


def kernel(x, edge_index, cluster, Wl0, bl0, Wr0, Wl1, bl1, Wr1):
    raise NotImplementedError("write your pallas kernel here")


# speedup vs baseline: 5.1552x; 5.1552x over previous
import jax
import jax.numpy as jnp
from jax import lax
from jax.experimental import pallas as pl
from jax.experimental.pallas import tpu as pltpu
from jax.experimental.pallas import tpu_sc as plsc

N = 10000
E = 320000
D = 128
NCLUST = 1000
NCORES = 2
NSUB = 16
NW = NCORES * NSUB
NPAD = 10240
EPW = E // NW
CH = 80
NEC = EPW // CH
RPT = NPAD // NSUB
CPW = NPAD // NW
NGC = CPW // CH
DEGR = NPAD // D
ACG = 1024
RGT = ACG // NSUB
GROWS = 16
BR = 512
GRID = NPAD // BR


def _hist_inc(hist2d, idx_ref, k):
    vec = idx_ref[pl.ds(k * 16, 16)]
    cnt, last = plsc.scan_count(vec)
    plsc.addupdate_scatter(
        hist2d, [vec >> 7, vec & 127], cnt.astype(jnp.float32), mask=last)


def _agg0_body(x_hbm, src_hbm, dst_hbm, zero_hbm, out_hbm, deg_hbm,
               idx_s, idx_d, rows, deg2d, idx80, acc, acc_deg, sem):
    c = lax.axis_index("c")
    s = lax.axis_index("s")
    wid = c * NSUB + s
    pltpu.sync_copy(zero_hbm, acc.at[pl.ds(s * RPT, RPT)])
    pltpu.sync_copy(zero_hbm.at[pl.ds(0, DEGR)], deg2d)

    @pl.when(s < DEGR // 8)
    def _():
        pltpu.sync_copy(zero_hbm.at[pl.ds(0, 8)], acc_deg.at[pl.ds(s * 8, 8)])

    for k in range(CH // 16):
        idx80[pl.ds(k * 16, 16)] = lax.iota(jnp.int32, 16) + k * 16
    plsc.subcore_barrier()

    def body(i, carry):
        base = wid * EPW + i * CH
        pltpu.sync_copy(src_hbm.at[pl.ds(base, CH)], idx_s)
        pltpu.sync_copy(dst_hbm.at[pl.ds(base, CH)], idx_d)
        pltpu.async_copy(x_hbm.at[idx_s], rows, sem).wait()
        pltpu.sync_copy(rows, acc.at[idx_d], add=True)
        for k in range(CH // 16):
            _hist_inc(deg2d, idx_d, k)
        return carry
    lax.fori_loop(0, NEC, body, 0)
    pltpu.sync_copy(deg2d, acc_deg.at[idx80], add=True)
    plsc.subcore_barrier()
    pltpu.sync_copy(acc.at[pl.ds(s * RPT, RPT)], out_hbm.at[c, pl.ds(s * RPT, RPT)])

    @pl.when(s < DEGR // 8)
    def _():
        pltpu.sync_copy(acc_deg.at[pl.ds(s * 8, 8)], deg_hbm.at[c, pl.ds(s * 8, 8)])


_agg0_call = pl.kernel(
    _agg0_body,
    out_type=(
        jax.ShapeDtypeStruct((NCORES, NPAD, D), jnp.float32),
        jax.ShapeDtypeStruct((NCORES, DEGR, D), jnp.float32),
    ),
    mesh=plsc.VectorSubcoreMesh(core_axis_name="c", subcore_axis_name="s"),
    compiler_params=pltpu.CompilerParams(needs_layout_passes=False),
    scratch_types=[
        pltpu.VMEM((CH,), jnp.int32),
        pltpu.VMEM((CH,), jnp.int32),
        pltpu.VMEM((CH, D), jnp.float32),
        pltpu.VMEM((DEGR, D), jnp.float32),
        pltpu.VMEM((CH,), jnp.int32),
        pltpu.VMEM_SHARED((NPAD, D), jnp.float32),
        pltpu.VMEM_SHARED((DEGR, D), jnp.float32),
        pltpu.SemaphoreType.DMA,
    ],
)


def _agg1_body(h_hbm, src_hbm, dst_hbm, clu_hbm, zero_hbm,
               out_hbm, outg_hbm, gcnt_hbm,
               idx_s, idx_d, rows, gcnt2d, idx16, acc, accg, acc_gcnt, sem):
    c = lax.axis_index("c")
    s = lax.axis_index("s")
    wid = c * NSUB + s
    pltpu.sync_copy(zero_hbm, acc.at[pl.ds(s * RPT, RPT)])
    pltpu.sync_copy(zero_hbm.at[pl.ds(0, RGT)], accg.at[pl.ds(s * RGT, RGT)])
    pltpu.sync_copy(zero_hbm.at[pl.ds(0, GROWS)], gcnt2d)

    @pl.when(s == 0)
    def _():
        pltpu.sync_copy(zero_hbm.at[pl.ds(0, GROWS)], acc_gcnt)

    idx16[...] = lax.iota(jnp.int32, 16)
    plsc.subcore_barrier()

    def body(i, carry):
        base = wid * EPW + i * CH
        pltpu.sync_copy(src_hbm.at[pl.ds(base, CH)], idx_s)
        pltpu.sync_copy(dst_hbm.at[pl.ds(base, CH)], idx_d)
        pltpu.async_copy(h_hbm.at[idx_s], rows, sem).wait()
        pltpu.sync_copy(rows, acc.at[idx_d], add=True)
        return carry
    lax.fori_loop(0, NEC, body, 0)

    def gbody(j, carry):
        base = wid * CPW + j * CH
        pltpu.sync_copy(clu_hbm.at[pl.ds(base, CH)], idx_d)
        pltpu.sync_copy(h_hbm.at[pl.ds(base, CH)], rows)
        pltpu.sync_copy(rows, accg.at[idx_d], add=True)
        for k in range(CH // 16):
            _hist_inc(gcnt2d, idx_d, k)
        return carry
    lax.fori_loop(0, NGC, gbody, 0)
    pltpu.sync_copy(gcnt2d, acc_gcnt.at[idx16], add=True)
    plsc.subcore_barrier()
    pltpu.sync_copy(acc.at[pl.ds(s * RPT, RPT)], out_hbm.at[c, pl.ds(s * RPT, RPT)])
    pltpu.sync_copy(accg.at[pl.ds(s * RGT, RGT)], outg_hbm.at[c, pl.ds(s * RGT, RGT)])

    @pl.when(s == 0)
    def _():
        pltpu.sync_copy(acc_gcnt, gcnt_hbm.at[c])


_agg1_call = pl.kernel(
    _agg1_body,
    out_type=(
        jax.ShapeDtypeStruct((NCORES, NPAD, D), jnp.float32),
        jax.ShapeDtypeStruct((NCORES, ACG, D), jnp.float32),
        jax.ShapeDtypeStruct((NCORES, GROWS, D), jnp.float32),
    ),
    mesh=plsc.VectorSubcoreMesh(core_axis_name="c", subcore_axis_name="s"),
    compiler_params=pltpu.CompilerParams(needs_layout_passes=False),
    scratch_types=[
        pltpu.VMEM((CH,), jnp.int32),
        pltpu.VMEM((CH,), jnp.int32),
        pltpu.VMEM((CH, D), jnp.float32),
        pltpu.VMEM((GROWS, D), jnp.float32),
        pltpu.VMEM((16,), jnp.int32),
        pltpu.VMEM_SHARED((NPAD, D), jnp.float32),
        pltpu.VMEM_SHARED((ACG, D), jnp.float32),
        pltpu.VMEM_SHARED((GROWS, D), jnp.float32),
        pltpu.SemaphoreType.DMA,
    ],
)


def _conv0_body(p_ref, x_ref, dg_ref, wl_ref, bl_ref, wr_ref, out_ref, h_ref):
    p = p_ref[...]
    dg = dg_ref[...]
    deg = jnp.maximum(dg[0] + dg[1], 1.0)
    mean = (p[0] + p[1]) / deg
    out = (jnp.dot(mean, wl_ref[...], preferred_element_type=jnp.float32)
           + bl_ref[...]
           + jnp.dot(x_ref[...], wr_ref[...], preferred_element_type=jnp.float32))
    out_ref[...] = out
    h_ref[...] = jnp.maximum(out, 0.0)


def _conv1_body(p_ref, h_ref, dg_ref, wl_ref, bl_ref, wr_ref, y_ref):
    p = p_ref[...]
    dg = dg_ref[...]
    deg = jnp.maximum(dg[0] + dg[1], 1.0)
    mean = (p[0] + p[1]) / deg
    x2 = (jnp.dot(mean, wl_ref[...], preferred_element_type=jnp.float32)
          + bl_ref[...]
          + jnp.dot(h_ref[...], wr_ref[...], preferred_element_type=jnp.float32))
    m = jnp.max(x2, axis=-1, keepdims=True)
    e = jnp.exp(x2 - m)
    lse = jnp.log(jnp.sum(e, axis=-1, keepdims=True))
    y_ref[...] = x2 - m - lse


def _pool_body(pg_ref, gc_ref, g_ref):
    pg = pg_ref[...]
    gc = gc_ref[...]
    cnt = jnp.maximum(gc[0] + gc[1], 1.0)
    g_ref[...] = (pg[0, :NCLUST] + pg[1, :NCLUST]) / cnt[:NCLUST]


def kernel(x, edge_index, cluster, Wl0, bl0, Wr0, Wl1, bl1, Wr1):
    f32 = jnp.float32
    src = edge_index[0]
    dst = edge_index[1]
    zeros_b = jnp.zeros((RPT, D), f32)

    p0, degp = _agg0_call(x, src, dst, zeros_b)
    degr = degp.reshape(NCORES, NPAD, 1)
    x_pad = jnp.pad(x, ((0, NPAD - N), (0, 0)))

    grid = (GRID,)
    out_full, h_full = pl.pallas_call(
        _conv0_body,
        grid=grid,
        in_specs=[
            pl.BlockSpec((NCORES, BR, D), lambda i: (0, i, 0)),
            pl.BlockSpec((BR, D), lambda i: (i, 0)),
            pl.BlockSpec((NCORES, BR, 1), lambda i: (0, i, 0)),
            pl.BlockSpec((D, D), lambda i: (0, 0)),
            pl.BlockSpec((1, D), lambda i: (0, 0)),
            pl.BlockSpec((D, D), lambda i: (0, 0)),
        ],
        out_specs=[
            pl.BlockSpec((BR, D), lambda i: (i, 0)),
            pl.BlockSpec((BR, D), lambda i: (i, 0)),
        ],
        out_shape=[
            jax.ShapeDtypeStruct((NPAD, D), f32),
            jax.ShapeDtypeStruct((NPAD, D), f32),
        ],
    )(p0, x_pad, degr, Wl0.T, bl0[None, :], Wr0.T)

    clu_pad = jnp.concatenate(
        [cluster, jnp.full((NPAD - N,), NCLUST, jnp.int32)], axis=0)

    p1, pg, gcntp = _agg1_call(h_full, src, dst, clu_pad, zeros_b)
    gcntr = gcntp.reshape(NCORES, GROWS * D, 1)

    y_full = pl.pallas_call(
        _conv1_body,
        grid=grid,
        in_specs=[
            pl.BlockSpec((NCORES, BR, D), lambda i: (0, i, 0)),
            pl.BlockSpec((BR, D), lambda i: (i, 0)),
            pl.BlockSpec((NCORES, BR, 1), lambda i: (0, i, 0)),
            pl.BlockSpec((D, D), lambda i: (0, 0)),
            pl.BlockSpec((1, D), lambda i: (0, 0)),
            pl.BlockSpec((D, D), lambda i: (0, 0)),
        ],
        out_specs=pl.BlockSpec((BR, D), lambda i: (i, 0)),
        out_shape=jax.ShapeDtypeStruct((NPAD, D), f32),
    )(p1, h_full, degr, Wl1.T, bl1[None, :], Wr1.T)

    g = pl.pallas_call(
        _pool_body,
        grid=(1,),
        in_specs=[
            pl.BlockSpec((NCORES, ACG, D), lambda i: (0, 0, 0)),
            pl.BlockSpec((NCORES, ACG, 1), lambda i: (0, 0, 0)),
        ],
        out_specs=pl.BlockSpec((NCLUST, D), lambda i: (0, 0)),
        out_shape=jax.ShapeDtypeStruct((NCLUST, D), f32),
    )(pg, gcntr)

    return (y_full[:N], out_full[:N], g)


# speedup vs baseline: 6.1987x; 1.2024x over previous
import jax
import jax.numpy as jnp
from jax import lax
from jax.experimental import pallas as pl
from jax.experimental.pallas import tpu as pltpu
from jax.experimental.pallas import tpu_sc as plsc

N = 10000
E = 320000
D = 128
NCLUST = 1000
NCORES = 2
NSUB = 16
NW = NCORES * NSUB
NPAD = 10240
ACCR = 10112
RPTA = ACCR // NSUB
EPW = E // NW
CH = 200
NEC = EPW // CH
NFV = CH // 16
DEGR = NPAD // D
BR = 400
GRID = N // BR


def _hist_chunk(hist2d, idx_c):
    for k in range(NFV + 1):
        off = k * 16 if k < NFV else CH - 16
        vec = idx_c[pl.ds(off, 16)]
        if k < NFV:
            mask = None
        else:
            mask = lax.iota(jnp.int32, 16) >= (NFV * 16 - (CH - 16))
        cnt, last = plsc.scan_count(vec, mask)
        if mask is not None:
            last = last & mask
        plsc.addupdate_scatter(
            hist2d, [vec >> 7, vec & 127], cnt.astype(jnp.float32), mask=last)


def _agg_body(feat_hbm, src_hbm, dst_hbm, zero_hbm, out_hbm, deg_hbm,
              src1d, idx_c, ring, deg2d, acc, semg, do_hist):
    c = lax.axis_index("c")
    s = lax.axis_index("s")
    wid = c * NSUB + s
    pltpu.sync_copy(zero_hbm.at[pl.ds(0, RPTA)], acc.at[pl.ds(s * RPTA, RPTA)])
    if do_hist:
        pltpu.sync_copy(zero_hbm.at[pl.ds(0, DEGR)], deg2d)
    pltpu.sync_copy(src_hbm.at[wid], src1d)
    plsc.subcore_barrier()

    def body(i, carry):
        gd = pltpu.async_copy(feat_hbm.at[src1d.at[pl.ds(i * CH, CH)]], ring, semg)
        pltpu.sync_copy(dst_hbm.at[pl.ds(wid * EPW + i * CH, CH)], idx_c)
        if do_hist:
            _hist_chunk(deg2d, idx_c)
        gd.wait()
        pltpu.sync_copy(ring, acc.at[idx_c], add=True)
        return carry
    lax.fori_loop(0, NEC, body, 0)

    if do_hist:
        pltpu.sync_copy(deg2d, deg_hbm.at[wid])
    plsc.subcore_barrier()
    pltpu.sync_copy(acc.at[pl.ds(s * RPTA, RPTA)], out_hbm.at[c, pl.ds(s * RPTA, RPTA)])


def _mk_agg(do_hist):
    outs = [jax.ShapeDtypeStruct((NCORES, ACCR, D), jnp.float32)]
    scratch = [
        pltpu.VMEM((EPW,), jnp.int32),
        pltpu.VMEM((CH,), jnp.int32),
        pltpu.VMEM((CH, D), jnp.float32),
        pltpu.VMEM((DEGR, D), jnp.float32),
        pltpu.VMEM_SHARED((ACCR, D), jnp.float32),
        pltpu.SemaphoreType.DMA,
    ]
    if do_hist:
        outs.append(jax.ShapeDtypeStruct((NW, DEGR, D), jnp.float32))

    def body(*refs):
        if do_hist:
            (feat_hbm, src_hbm, dst_hbm, zero_hbm, out_hbm, deg_hbm,
             src1d, idx_c, ring, deg2d, acc, semg) = refs
        else:
            (feat_hbm, src_hbm, dst_hbm, zero_hbm, out_hbm,
             src1d, idx_c, ring, deg2d, acc, semg) = refs
            deg_hbm = None
        _agg_body(feat_hbm, src_hbm, dst_hbm, zero_hbm, out_hbm, deg_hbm,
                  src1d, idx_c, ring, deg2d, acc, semg, do_hist)

    return pl.kernel(
        body,
        out_type=tuple(outs) if do_hist else outs[0],
        mesh=plsc.VectorSubcoreMesh(core_axis_name="c", subcore_axis_name="s"),
        compiler_params=pltpu.CompilerParams(needs_layout_passes=False),
        scratch_types=scratch,
    )


_agg0_call = _mk_agg(True)
_agg1_call = _mk_agg(False)


def _conv0_body(p_ref, x_ref, dg_ref, wl_ref, bl_ref, wr_ref, out_ref, h_ref):
    p = p_ref[...]
    deg = jnp.maximum(jnp.sum(dg_ref[...], axis=0), 1.0)
    mean = (p[0] + p[1]) / deg
    out = (jnp.dot(mean, wl_ref[...], preferred_element_type=jnp.float32)
           + bl_ref[...]
           + jnp.dot(x_ref[...], wr_ref[...], preferred_element_type=jnp.float32))
    out_ref[...] = out
    h_ref[...] = jnp.maximum(out, 0.0)


def _conv1_body(p_ref, h_ref, dg_ref, wl_ref, bl_ref, wr_ref, y_ref):
    p = p_ref[...]
    deg = jnp.maximum(jnp.sum(dg_ref[...], axis=0), 1.0)
    mean = (p[0] + p[1]) / deg
    x2 = (jnp.dot(mean, wl_ref[...], preferred_element_type=jnp.float32)
          + bl_ref[...]
          + jnp.dot(h_ref[...], wr_ref[...], preferred_element_type=jnp.float32))
    m = jnp.max(x2, axis=-1, keepdims=True)
    e = jnp.exp(x2 - m)
    lse = jnp.log(jnp.sum(e, axis=-1, keepdims=True))
    y_ref[...] = x2 - m - lse


def _pool_body(clu_ref, h_ref, g_ref, cnt_ref):
    i = pl.program_id(0)

    @pl.when(i == 0)
    def _():
        g_ref[...] = jnp.zeros((NCLUST, D), jnp.float32)
        cnt_ref[...] = jnp.zeros((NCLUST, 1), jnp.float32)

    mt = (lax.broadcasted_iota(jnp.int32, (NCLUST, BR), 0)
          == clu_ref[0]).astype(jnp.float32)
    g_ref[...] += jnp.dot(mt, h_ref[...], preferred_element_type=jnp.float32)
    cnt_ref[...] += jnp.sum(mt, axis=1, keepdims=True)

    @pl.when(i == GRID - 1)
    def _():
        g_ref[...] = g_ref[...] / jnp.maximum(cnt_ref[...], 1.0)


def kernel(x, edge_index, cluster, Wl0, bl0, Wr0, Wl1, bl1, Wr1):
    f32 = jnp.float32
    src2 = edge_index[0].reshape(NW, EPW)
    dst1 = edge_index[1]
    zeros_b = jnp.zeros((RPTA, D), f32)

    p0, degp = _agg0_call(x, src2, dst1, zeros_b)
    degr = degp.reshape(NW, NPAD, 1)

    grid = (GRID,)
    out, h = pl.pallas_call(
        _conv0_body,
        grid=grid,
        in_specs=[
            pl.BlockSpec((NCORES, BR, D), lambda i: (0, i, 0)),
            pl.BlockSpec((BR, D), lambda i: (i, 0)),
            pl.BlockSpec((NW, BR, 1), lambda i: (0, i, 0)),
            pl.BlockSpec((D, D), lambda i: (0, 0)),
            pl.BlockSpec((1, D), lambda i: (0, 0)),
            pl.BlockSpec((D, D), lambda i: (0, 0)),
        ],
        out_specs=[
            pl.BlockSpec((BR, D), lambda i: (i, 0)),
            pl.BlockSpec((BR, D), lambda i: (i, 0)),
        ],
        out_shape=[
            jax.ShapeDtypeStruct((N, D), f32),
            jax.ShapeDtypeStruct((N, D), f32),
        ],
    )(p0, x, degr, Wl0.T, bl0[None, :], Wr0.T)

    p1 = _agg1_call(h, src2, dst1, zeros_b)

    y = pl.pallas_call(
        _conv1_body,
        grid=grid,
        in_specs=[
            pl.BlockSpec((NCORES, BR, D), lambda i: (0, i, 0)),
            pl.BlockSpec((BR, D), lambda i: (i, 0)),
            pl.BlockSpec((NW, BR, 1), lambda i: (0, i, 0)),
            pl.BlockSpec((D, D), lambda i: (0, 0)),
            pl.BlockSpec((1, D), lambda i: (0, 0)),
            pl.BlockSpec((D, D), lambda i: (0, 0)),
        ],
        out_specs=pl.BlockSpec((BR, D), lambda i: (i, 0)),
        out_shape=jax.ShapeDtypeStruct((N, D), f32),
    )(p1, h, degr, Wl1.T, bl1[None, :], Wr1.T)

    clu2 = cluster.reshape(GRID, 1, BR)
    g = pl.pallas_call(
        _pool_body,
        grid=grid,
        in_specs=[
            pl.BlockSpec((1, 1, BR), lambda i: (i, 0, 0)),
            pl.BlockSpec((BR, D), lambda i: (i, 0)),
        ],
        out_specs=pl.BlockSpec((NCLUST, D), lambda i: (0, 0)),
        out_shape=jax.ShapeDtypeStruct((NCLUST, D), f32),
        scratch_shapes=[pltpu.VMEM((NCLUST, 1), f32)],
    )(clu2, h)

    return (y, out, g)
